# baseline (speedup 1.0000x reference)
import jax
import jax.numpy as jnp
from jax.experimental import pallas as pl
from jax.experimental.pallas import tpu as pltpu

_F = 16
_KT = 50
_P = 13
_EPS = 1e-5
_NS = (_P - 1 + _KT - 1) // _P + 1


def _conv_body(x_ref, wa_ref, affa_ref, wb_ref, affb_ref, perm_ref, o_ref,
               xw_ref, zb_ref, zw_ref):
    ga = o_ref.shape[0]
    kc = x_ref.shape[2]
    c = kc // _P
    f = o_ref.shape[2]
    na = c * f
    kf = _P * f
    m2 = zb_ref.shape[0] // (ga * c)
    p1 = m2 * _P
    p2 = o_ref.shape[1] // c
    rows = xw_ref.shape[0] // ga
    tail_a = xw_ref.shape[1] - (_NS - 1) * kc
    tail_b = zw_ref.shape[1] - (_NS - 1) * kf

    for g in range(ga):
        r0 = g * rows
        for a in range(_NS - 1):
            xw_ref[r0:r0 + p1, a * kc:(a + 1) * kc] = x_ref[g, a:a + p1, :]
        xw_ref[r0:r0 + p1, (_NS - 1) * kc:] = \
            x_ref[g, _NS - 1:_NS - 1 + p1, 0:tail_a]
        xw_ref[r0 + p1:r0 + rows, :] = jnp.zeros_like(xw_ref[r0 + p1:r0 + rows])

    acc = jnp.dot(xw_ref[...], wa_ref[...], preferred_element_type=jnp.float32)
    m = acc[:, 0:na]
    for r in range(1, _P):
        m = jnp.maximum(m, acc[:, r * na:(r + 1) * na])
    y = jnp.maximum(m + affa_ref[0:1], 0.0) * affa_ref[1:2] + affa_ref[2:3]
    y = y.astype(zb_ref.dtype)

    for g in range(ga):
        yp = jnp.dot(perm_ref[...], y[g * rows:(g + 1) * rows],
                     preferred_element_type=jnp.float32).astype(zb_ref.dtype)
        for ci in range(c):
            zb0 = (g * c + ci) * m2
            for s in range(_P):
                zb_ref[zb0:zb0 + m2, s * f:(s + 1) * f] = \
                    yp[s * m2:(s + 1) * m2, ci * f:(ci + 1) * f]

    for gc in range(ga * c):
        r0 = gc * p2
        z0 = gc * m2
        for a in range(_NS - 1):
            zw_ref[r0:r0 + p2, a * kf:(a + 1) * kf] = \
                zb_ref[z0 + a:z0 + a + p2, :]
        zw_ref[r0:r0 + p2, (_NS - 1) * kf:] = \
            zb_ref[z0 + _NS - 1:z0 + _NS - 1 + p2, 0:tail_b]

    accb = jnp.dot(zw_ref[...], wb_ref[...],
                   preferred_element_type=jnp.float32)
    mb = accb[:, 0:f]
    for r in range(1, _P):
        mb = jnp.maximum(mb, accb[:, r * f:(r + 1) * f])
    z2 = jnp.maximum(mb + affb_ref[0:1], 0.0) * affb_ref[1:2] + affb_ref[2:3]
    z2 = z2.astype(o_ref.dtype)
    for g in range(ga):
        o_ref[g] = z2[g * c * p2:(g + 1) * c * p2]


def _head_body(z_ref, w_ref, b_ref, o_ref):
    o_ref[...] = (jnp.dot(z_ref[...], w_ref[...],
                          preferred_element_type=jnp.float32) + b_ref[...])


def kernel(x, w1, b1, w2, b2, w3, b3, gamma1, beta1, mean1, var1,
           gamma2, beta2, mean2, var2, w_lin, b_lin):
    B, T, C = x.shape
    F = _F
    T2 = T - (_KT - 1)
    P1 = (T2 - _P) // _P + 1
    T3 = P1 - (_KT - 1)
    P2 = (T3 - _P) // _P + 1
    E = w_lin.shape[0]
    M1 = P1 + _NS - 1
    M2 = P2 + _NS - 1

    s1 = gamma1 * jax.lax.rsqrt(var1 + _EPS)
    t1 = beta1 - mean1 * s1
    s2 = gamma2 * jax.lax.rsqrt(var2 + _EPS)
    t2 = beta2 - mean2 * s2

    a_i = jnp.arange(_NS)[:, None, None]
    s_i = jnp.arange(_P)[None, :, None]
    r_i = jnp.arange(_P)[None, None, :]
    tap = _P * a_i + s_i - r_i
    ok = (tap >= 0) & (tap < _KT)
    tap_c = jnp.clip(tap, 0, _KT - 1)

    win_tail = (_P - 1 + _KT - 1 + 1) - (_NS - 1) * _P
    ka = ((_NS - 1) * _P + win_tail) * C
    kb = ((_NS - 1) * _P + win_tail) * F

    w2_band = jnp.where(ok[..., None], w2.T[tap_c], 0.0)
    w_a = jnp.einsum("ci,asrf->asircf", w1, w2_band)
    w_a = w_a.reshape(_NS * _P * C, _P * C * F)[:ka].astype(jnp.bfloat16)
    bias_a = (b1[:, None] * jnp.sum(w2, axis=1)[None, :]
              + b2[None, :]).reshape(-1)
    aff_a = jnp.stack([bias_a, jnp.tile(s1, C), jnp.tile(t1, C)], axis=0)

    w3_band = jnp.where(ok[..., None, None],
                        jnp.transpose(w3, (2, 1, 0))[tap_c], 0.0)
    w_b = (jnp.transpose(w3_band, (0, 1, 3, 2, 4))
           .reshape(_NS * _P * F, _P * F)[:kb].astype(jnp.bfloat16))
    aff_b = jnp.stack([b3, s2, t2], axis=0)

    w_l = (w_lin.reshape(E, F, P2, C).transpose(3, 2, 1, 0)
           .reshape(C * P2 * F, E).astype(jnp.bfloat16))
    b_l = b_lin.reshape(1, E)

    xb = (jnp.pad(x, ((0, 0), (0, M1 * _P - T), (0, 0)))
          .reshape(B, M1, _P * C).astype(jnp.bfloat16))

    ga = 8
    while B % ga:
        ga //= 2
    p1r = ((M2 * _P + 7) // 8) * 8
    s_t = jnp.arange(_P * M2) // M2
    m_t = jnp.arange(_P * M2) % M2
    perm = (jnp.zeros((p1r, p1r), jnp.bfloat16)
            .at[jnp.arange(_P * M2), _P * m_t + s_t].set(1))
    z2 = pl.pallas_call(
        _conv_body,
        out_shape=jax.ShapeDtypeStruct((B, C * P2, F), jnp.bfloat16),
        grid=(B // ga,),
        in_specs=[
            pl.BlockSpec((ga, M1, _P * C), lambda i: (i, 0, 0)),
            pl.BlockSpec(w_a.shape, lambda i: (0, 0)),
            pl.BlockSpec(aff_a.shape, lambda i: (0, 0)),
            pl.BlockSpec(w_b.shape, lambda i: (0, 0)),
            pl.BlockSpec(aff_b.shape, lambda i: (0, 0)),
            pl.BlockSpec(perm.shape, lambda i: (0, 0)),
        ],
        out_specs=pl.BlockSpec((ga, C * P2, F), lambda i: (i, 0, 0)),
        scratch_shapes=[
            pltpu.VMEM((ga * p1r, ka), jnp.bfloat16),
            pltpu.VMEM((ga * C * M2, _P * F), jnp.bfloat16),
            pltpu.VMEM((ga * C * P2, kb), jnp.bfloat16),
        ],
        compiler_params=pltpu.CompilerParams(
            dimension_semantics=("parallel",)),
    )(xb, w_a, aff_a, w_b, aff_b, perm)

    zf = z2.reshape(B, C * P2 * F)

    bb = min(B, 128)
    out = pl.pallas_call(
        _head_body,
        out_shape=jax.ShapeDtypeStruct((B, E), jnp.float32),
        grid=(pl.cdiv(B, bb),),
        in_specs=[
            pl.BlockSpec((bb, C * P2 * F), lambda i: (i, 0)),
            pl.BlockSpec(w_l.shape, lambda i: (0, 0)),
            pl.BlockSpec(b_l.shape, lambda i: (0, 0)),
        ],
        out_specs=pl.BlockSpec((bb, E), lambda i: (i, 0)),
        compiler_params=pltpu.CompilerParams(
            dimension_semantics=("parallel",)),
    )(zf, w_l, b_l)
    return out


# speedup vs baseline: 1.0224x; 1.0224x over previous
import jax
import jax.numpy as jnp
from jax.experimental import pallas as pl
from jax.experimental.pallas import tpu as pltpu

_F = 16
_KT = 50
_P = 13
_EPS = 1e-5
_NS = (_P - 1 + _KT - 1) // _P + 1


def _conv_body(x_ref, wa_ref, affa_ref, wb_ref, affb_ref, perm_ref, o_ref,
               xw_ref, zb_ref, zw_ref):
    ga = o_ref.shape[0]
    kc = x_ref.shape[2]
    c = kc // _P
    f = o_ref.shape[2]
    na = c * f
    kf = _P * f
    m2 = zb_ref.shape[0] // (ga * c)
    p1 = m2 * _P
    p2 = o_ref.shape[1] // c
    rows = xw_ref.shape[0] // ga
    tail_a = xw_ref.shape[1] - (_NS - 1) * kc
    tail_b = zw_ref.shape[1] - (_NS - 1) * kf

    for g in range(ga):
        r0 = g * rows
        for a in range(_NS - 1):
            xw_ref[r0:r0 + p1, a * kc:(a + 1) * kc] = x_ref[g, a:a + p1, :]
        xw_ref[r0:r0 + p1, (_NS - 1) * kc:] = \
            x_ref[g, _NS - 1:_NS - 1 + p1, 0:tail_a]
        xw_ref[r0 + p1:r0 + rows, :] = jnp.zeros_like(xw_ref[r0 + p1:r0 + rows])

    chunks = []
    rem = _P
    while rem > 3:
        chunks.append(2)
        rem -= 2
    chunks.append(rem)
    m = None
    n0 = 0
    for k in chunks:
        d = jnp.dot(xw_ref[...], wa_ref[:, n0:n0 + k * na],
                    preferred_element_type=jnp.float32)
        for l in range(k):
            piece = d[:, l * na:(l + 1) * na]
            m = piece if m is None else jnp.maximum(m, piece)
        n0 += k * na
    y = jnp.maximum(m + affa_ref[0:1], 0.0) * affa_ref[1:2] + affa_ref[2:3]
    y = y.astype(zb_ref.dtype)

    for g in range(ga):
        yp = jnp.dot(perm_ref[...], y[g * rows:(g + 1) * rows],
                     preferred_element_type=jnp.float32).astype(zb_ref.dtype)
        for ci in range(c):
            zb0 = (g * c + ci) * m2
            for s in range(_P):
                zb_ref[zb0:zb0 + m2, s * f:(s + 1) * f] = \
                    yp[s * m2:(s + 1) * m2, ci * f:(ci + 1) * f]

    for gc in range(ga * c):
        r0 = gc * p2
        z0 = gc * m2
        for a in range(_NS - 1):
            zw_ref[r0:r0 + p2, a * kf:(a + 1) * kf] = \
                zb_ref[z0 + a:z0 + a + p2, :]
        zw_ref[r0:r0 + p2, (_NS - 1) * kf:] = \
            zb_ref[z0 + _NS - 1:z0 + _NS - 1 + p2, 0:tail_b]

    accb = jnp.dot(zw_ref[...], wb_ref[...],
                   preferred_element_type=jnp.float32)
    mb = accb[:, 0:f]
    for r in range(1, _P):
        mb = jnp.maximum(mb, accb[:, r * f:(r + 1) * f])
    z2 = jnp.maximum(mb + affb_ref[0:1], 0.0) * affb_ref[1:2] + affb_ref[2:3]
    z2 = z2.astype(o_ref.dtype)
    for g in range(ga):
        o_ref[g] = z2[g * c * p2:(g + 1) * c * p2]


def _head_body(z_ref, w_ref, b_ref, o_ref):
    o_ref[...] = (jnp.dot(z_ref[...], w_ref[...],
                          preferred_element_type=jnp.float32) + b_ref[...])


def kernel(x, w1, b1, w2, b2, w3, b3, gamma1, beta1, mean1, var1,
           gamma2, beta2, mean2, var2, w_lin, b_lin):
    B, T, C = x.shape
    F = _F
    T2 = T - (_KT - 1)
    P1 = (T2 - _P) // _P + 1
    T3 = P1 - (_KT - 1)
    P2 = (T3 - _P) // _P + 1
    E = w_lin.shape[0]
    M1 = P1 + _NS - 1
    M2 = P2 + _NS - 1

    s1 = gamma1 * jax.lax.rsqrt(var1 + _EPS)
    t1 = beta1 - mean1 * s1
    s2 = gamma2 * jax.lax.rsqrt(var2 + _EPS)
    t2 = beta2 - mean2 * s2

    a_i = jnp.arange(_NS)[:, None, None]
    s_i = jnp.arange(_P)[None, :, None]
    r_i = jnp.arange(_P)[None, None, :]
    tap = _P * a_i + s_i - r_i
    ok = (tap >= 0) & (tap < _KT)
    tap_c = jnp.clip(tap, 0, _KT - 1)

    win_tail = (_P - 1 + _KT - 1 + 1) - (_NS - 1) * _P
    ka = ((_NS - 1) * _P + win_tail) * C
    kb = ((_NS - 1) * _P + win_tail) * F

    w2_band = jnp.where(ok[..., None], w2.T[tap_c], 0.0)
    w_a = jnp.einsum("ci,asrf->asircf", w1, w2_band)
    w_a = w_a.reshape(_NS * _P * C, _P * C * F)[:ka].astype(jnp.bfloat16)
    bias_a = (b1[:, None] * jnp.sum(w2, axis=1)[None, :]
              + b2[None, :]).reshape(-1)
    aff_a = jnp.stack([bias_a, jnp.tile(s1, C), jnp.tile(t1, C)], axis=0)

    w3_band = jnp.where(ok[..., None, None],
                        jnp.transpose(w3, (2, 1, 0))[tap_c], 0.0)
    w_b = (jnp.transpose(w3_band, (0, 1, 3, 2, 4))
           .reshape(_NS * _P * F, _P * F)[:kb].astype(jnp.bfloat16))
    aff_b = jnp.stack([b3, s2, t2], axis=0)

    w_l = (w_lin.reshape(E, F, P2, C).transpose(3, 2, 1, 0)
           .reshape(C * P2 * F, E).astype(jnp.bfloat16))
    b_l = b_lin.reshape(1, E)

    xb = (jnp.pad(x, ((0, 0), (0, M1 * _P - T), (0, 0)))
          .reshape(B, M1, _P * C).astype(jnp.bfloat16))

    ga = 16
    while B % ga:
        ga //= 2
    p1r = ((M2 * _P + 7) // 8) * 8
    s_t = jnp.arange(_P * M2) // M2
    m_t = jnp.arange(_P * M2) % M2
    perm = (jnp.zeros((p1r, p1r), jnp.bfloat16)
            .at[jnp.arange(_P * M2), _P * m_t + s_t].set(1))
    z2 = pl.pallas_call(
        _conv_body,
        out_shape=jax.ShapeDtypeStruct((B, C * P2, F), jnp.bfloat16),
        grid=(B // ga,),
        in_specs=[
            pl.BlockSpec((ga, M1, _P * C), lambda i: (i, 0, 0)),
            pl.BlockSpec(w_a.shape, lambda i: (0, 0)),
            pl.BlockSpec(aff_a.shape, lambda i: (0, 0)),
            pl.BlockSpec(w_b.shape, lambda i: (0, 0)),
            pl.BlockSpec(aff_b.shape, lambda i: (0, 0)),
            pl.BlockSpec(perm.shape, lambda i: (0, 0)),
        ],
        out_specs=pl.BlockSpec((ga, C * P2, F), lambda i: (i, 0, 0)),
        scratch_shapes=[
            pltpu.VMEM((ga * p1r, ka), jnp.bfloat16),
            pltpu.VMEM((ga * C * M2, _P * F), jnp.bfloat16),
            pltpu.VMEM((ga * C * P2, kb), jnp.bfloat16),
        ],
        compiler_params=pltpu.CompilerParams(
            dimension_semantics=("parallel",)),
    )(xb, w_a, aff_a, w_b, aff_b, perm)

    zf = z2.reshape(B, C * P2 * F)

    bb = min(B, 128)
    out = pl.pallas_call(
        _head_body,
        out_shape=jax.ShapeDtypeStruct((B, E), jnp.float32),
        grid=(pl.cdiv(B, bb),),
        in_specs=[
            pl.BlockSpec((bb, C * P2 * F), lambda i: (i, 0)),
            pl.BlockSpec(w_l.shape, lambda i: (0, 0)),
            pl.BlockSpec(b_l.shape, lambda i: (0, 0)),
        ],
        out_specs=pl.BlockSpec((bb, E), lambda i: (i, 0)),
        compiler_params=pltpu.CompilerParams(
            dimension_semantics=("parallel",)),
    )(zf, w_l, b_l)
    return out


# speedup vs baseline: 1.1822x; 1.1563x over previous
import jax
import jax.numpy as jnp
from jax.experimental import pallas as pl
from jax.experimental.pallas import tpu as pltpu

_F = 16
_KT = 50
_P = 13
_EPS = 1e-5
_NS = (_P - 1 + _KT - 1) // _P + 1


def _conv_body(x_ref, wa_ref, affa_ref, wb_ref, affb_ref, perm_ref, o_ref,
               xw_ref, zb_ref, zw_ref):
    ga = o_ref.shape[0]
    kc = x_ref.shape[2]
    c = kc // _P
    f = o_ref.shape[2]
    na = c * f
    kf = _P * f
    m2 = zb_ref.shape[0] // (ga * c)
    p1 = m2 * _P
    p2 = o_ref.shape[1] // c
    rows = xw_ref.shape[0] // ga
    tail_a = xw_ref.shape[1] - (_NS - 1) * kc
    tail_b = zw_ref.shape[1] - (_NS - 1) * kf

    for g in range(ga):
        r0 = g * rows
        for a in range(_NS - 1):
            xw_ref[r0:r0 + p1, a * kc:(a + 1) * kc] = x_ref[g, a:a + p1, :]
        xw_ref[r0:r0 + p1, (_NS - 1) * kc:] = \
            x_ref[g, _NS - 1:_NS - 1 + p1, 0:tail_a]
        xw_ref[r0 + p1:r0 + rows, :] = jnp.zeros_like(xw_ref[r0 + p1:r0 + rows])

    chunks = []
    rem = _P
    while rem > 3:
        chunks.append(2)
        rem -= 2
    chunks.append(rem)
    for g in range(ga):
        xg = xw_ref[g * rows:(g + 1) * rows, :]
        m = None
        n0 = 0
        for k in chunks:
            d = jnp.dot(xg, wa_ref[:, n0:n0 + k * na],
                        preferred_element_type=jnp.float32)
            for l in range(k):
                piece = d[:, l * na:(l + 1) * na]
                m = piece if m is None else jnp.maximum(m, piece)
            n0 += k * na
        y = jnp.maximum(m + affa_ref[0:1], 0.0) * affa_ref[1:2] + affa_ref[2:3]
        y = y.astype(zb_ref.dtype)

        yp = jnp.dot(perm_ref[...], y,
                     preferred_element_type=jnp.float32).astype(zb_ref.dtype)
        for ci in range(c):
            zb0 = (g * c + ci) * m2
            for s in range(_P):
                zb_ref[zb0:zb0 + m2, s * f:(s + 1) * f] = \
                    yp[s * m2:(s + 1) * m2, ci * f:(ci + 1) * f]

    for gc in range(ga * c):
        r0 = gc * p2
        z0 = gc * m2
        for a in range(_NS - 1):
            zw_ref[r0:r0 + p2, a * kf:(a + 1) * kf] = \
                zb_ref[z0 + a:z0 + a + p2, :]
        zw_ref[r0:r0 + p2, (_NS - 1) * kf:] = \
            zb_ref[z0 + _NS - 1:z0 + _NS - 1 + p2, 0:tail_b]

    for g in range(ga):
        accb = jnp.dot(zw_ref[g * c * p2:(g + 1) * c * p2, :], wb_ref[...],
                       preferred_element_type=jnp.float32)
        mb = accb[:, 0:f]
        for r in range(1, _P):
            mb = jnp.maximum(mb, accb[:, r * f:(r + 1) * f])
        z2 = (jnp.maximum(mb + affb_ref[0:1], 0.0) * affb_ref[1:2]
              + affb_ref[2:3])
        o_ref[g] = z2.astype(o_ref.dtype)


def _head_body(z_ref, w_ref, b_ref, o_ref):
    o_ref[...] = (jnp.dot(z_ref[...], w_ref[...],
                          preferred_element_type=jnp.float32) + b_ref[...])


def kernel(x, w1, b1, w2, b2, w3, b3, gamma1, beta1, mean1, var1,
           gamma2, beta2, mean2, var2, w_lin, b_lin):
    B, T, C = x.shape
    F = _F
    T2 = T - (_KT - 1)
    P1 = (T2 - _P) // _P + 1
    T3 = P1 - (_KT - 1)
    P2 = (T3 - _P) // _P + 1
    E = w_lin.shape[0]
    M1 = P1 + _NS - 1
    M2 = P2 + _NS - 1

    s1 = gamma1 * jax.lax.rsqrt(var1 + _EPS)
    t1 = beta1 - mean1 * s1
    s2 = gamma2 * jax.lax.rsqrt(var2 + _EPS)
    t2 = beta2 - mean2 * s2

    a_i = jnp.arange(_NS)[:, None, None]
    s_i = jnp.arange(_P)[None, :, None]
    r_i = jnp.arange(_P)[None, None, :]
    tap = _P * a_i + s_i - r_i
    ok = (tap >= 0) & (tap < _KT)
    tap_c = jnp.clip(tap, 0, _KT - 1)

    win_tail = (_P - 1 + _KT - 1 + 1) - (_NS - 1) * _P
    ka = ((_NS - 1) * _P + win_tail) * C
    kb = ((_NS - 1) * _P + win_tail) * F

    w2_band = jnp.where(ok[..., None], w2.T[tap_c], 0.0)
    w_a = jnp.einsum("ci,asrf->asircf", w1, w2_band)
    w_a = w_a.reshape(_NS * _P * C, _P * C * F)[:ka].astype(jnp.bfloat16)
    bias_a = (b1[:, None] * jnp.sum(w2, axis=1)[None, :]
              + b2[None, :]).reshape(-1)
    aff_a = jnp.stack([bias_a, jnp.tile(s1, C), jnp.tile(t1, C)], axis=0)

    w3_band = jnp.where(ok[..., None, None],
                        jnp.transpose(w3, (2, 1, 0))[tap_c], 0.0)
    w_b = (jnp.transpose(w3_band, (0, 1, 3, 2, 4))
           .reshape(_NS * _P * F, _P * F)[:kb].astype(jnp.bfloat16))
    aff_b = jnp.stack([b3, s2, t2], axis=0)

    w_l = (w_lin.reshape(E, F, P2, C).transpose(3, 2, 1, 0)
           .reshape(C * P2 * F, E).astype(jnp.bfloat16))
    b_l = b_lin.reshape(1, E)

    xb = (jnp.pad(x, ((0, 0), (0, M1 * _P - T), (0, 0)))
          .reshape(B, M1, _P * C).astype(jnp.bfloat16))

    ga = 16
    while B % ga:
        ga //= 2
    p1r = ((M2 * _P + 7) // 8) * 8
    s_t = jnp.arange(_P * M2) // M2
    m_t = jnp.arange(_P * M2) % M2
    perm = (jnp.zeros((p1r, p1r), jnp.bfloat16)
            .at[jnp.arange(_P * M2), _P * m_t + s_t].set(1))
    z2 = pl.pallas_call(
        _conv_body,
        out_shape=jax.ShapeDtypeStruct((B, C * P2, F), jnp.bfloat16),
        grid=(B // ga,),
        in_specs=[
            pl.BlockSpec((ga, M1, _P * C), lambda i: (i, 0, 0)),
            pl.BlockSpec(w_a.shape, lambda i: (0, 0)),
            pl.BlockSpec(aff_a.shape, lambda i: (0, 0)),
            pl.BlockSpec(w_b.shape, lambda i: (0, 0)),
            pl.BlockSpec(aff_b.shape, lambda i: (0, 0)),
            pl.BlockSpec(perm.shape, lambda i: (0, 0)),
        ],
        out_specs=pl.BlockSpec((ga, C * P2, F), lambda i: (i, 0, 0)),
        scratch_shapes=[
            pltpu.VMEM((ga * p1r, ka), jnp.bfloat16),
            pltpu.VMEM((ga * C * M2, _P * F), jnp.bfloat16),
            pltpu.VMEM((ga * C * P2, kb), jnp.bfloat16),
        ],
        compiler_params=pltpu.CompilerParams(
            dimension_semantics=("parallel",)),
    )(xb, w_a, aff_a, w_b, aff_b, perm)

    zf = z2.reshape(B, C * P2 * F)

    bb = min(B, 128)
    out = pl.pallas_call(
        _head_body,
        out_shape=jax.ShapeDtypeStruct((B, E), jnp.float32),
        grid=(pl.cdiv(B, bb),),
        in_specs=[
            pl.BlockSpec((bb, C * P2 * F), lambda i: (i, 0)),
            pl.BlockSpec(w_l.shape, lambda i: (0, 0)),
            pl.BlockSpec(b_l.shape, lambda i: (0, 0)),
        ],
        out_specs=pl.BlockSpec((bb, E), lambda i: (i, 0)),
        compiler_params=pltpu.CompilerParams(
            dimension_semantics=("parallel",)),
    )(zf, w_l, b_l)
    return out


# speedup vs baseline: 1.2113x; 1.0246x over previous
import jax
import jax.numpy as jnp
from jax.experimental import pallas as pl
from jax.experimental.pallas import tpu as pltpu

_F = 16
_KT = 50
_P = 13
_EPS = 1e-5
_NS = (_P - 1 + _KT - 1) // _P + 1


def _conv_body(x_ref, wa_ref, affa_ref, wb_ref, affb_ref, perm_ref, o_ref,
               xw_ref, zb_ref, zw_ref):
    ga = o_ref.shape[0]
    kc = x_ref.shape[2]
    c = kc // _P
    f = o_ref.shape[2]
    na = c * f
    kf = _P * f
    p2 = o_ref.shape[1] // c
    m2 = p2 + _NS - 1
    p1 = m2 * _P
    zbs = zb_ref.shape[0] // (ga * c)
    m2r = perm_ref.shape[0] // _P
    rows = xw_ref.shape[0] // ga
    tail_a = xw_ref.shape[1] - (_NS - 1) * kc
    tail_b = zw_ref.shape[1] - (_NS - 1) * kf

    for g in range(ga):
        r0 = g * rows
        for a in range(_NS - 1):
            xw_ref[r0:r0 + p1, a * kc:(a + 1) * kc] = x_ref[g, a:a + p1, :]
        xw_ref[r0:r0 + p1, (_NS - 1) * kc:] = \
            x_ref[g, _NS - 1:_NS - 1 + p1, 0:tail_a]
        xw_ref[r0 + p1:r0 + rows, :] = jnp.zeros_like(xw_ref[r0 + p1:r0 + rows])

    chunks = []
    rem = _P
    while rem > 3:
        chunks.append(2)
        rem -= 2
    chunks.append(rem)
    for g in range(ga):
        xg = xw_ref[g * rows:(g + 1) * rows, :]
        m = None
        n0 = 0
        for k in chunks:
            d = jnp.dot(xg, wa_ref[:, n0:n0 + k * na],
                        preferred_element_type=jnp.float32)
            for l in range(k):
                piece = d[:, l * na:(l + 1) * na]
                m = piece if m is None else jnp.maximum(m, piece)
            n0 += k * na
        y = jnp.maximum(m + affa_ref[0:1], 0.0) * affa_ref[1:2] + affa_ref[2:3]
        y = y.astype(zb_ref.dtype)

        yp = jnp.dot(perm_ref[...], y,
                     preferred_element_type=jnp.float32).astype(zb_ref.dtype)
        for ci in range(c):
            zb0 = (g * c + ci) * zbs
            for s in range(_P):
                zb_ref[zb0:zb0 + m2, s * f:(s + 1) * f] = \
                    yp[s * m2r:s * m2r + m2, ci * f:(ci + 1) * f]

    for gc in range(ga * c):
        r0 = gc * p2
        z0 = gc * zbs
        for a in range(_NS - 1):
            zw_ref[r0:r0 + p2, a * kf:(a + 1) * kf] = \
                zb_ref[z0 + a:z0 + a + p2, :]
        zw_ref[r0:r0 + p2, (_NS - 1) * kf:] = \
            zb_ref[z0 + _NS - 1:z0 + _NS - 1 + p2, 0:tail_b]

    for g in range(ga):
        accb = jnp.dot(zw_ref[g * c * p2:(g + 1) * c * p2, :], wb_ref[...],
                       preferred_element_type=jnp.float32)
        mb = accb[:, 0:f]
        for r in range(1, _P):
            mb = jnp.maximum(mb, accb[:, r * f:(r + 1) * f])
        z2 = (jnp.maximum(mb + affb_ref[0:1], 0.0) * affb_ref[1:2]
              + affb_ref[2:3])
        o_ref[g] = z2.astype(o_ref.dtype)


def _head_body(z_ref, w_ref, b_ref, o_ref):
    o_ref[...] = (jnp.dot(z_ref[...], w_ref[...],
                          preferred_element_type=jnp.float32) + b_ref[...])


def kernel(x, w1, b1, w2, b2, w3, b3, gamma1, beta1, mean1, var1,
           gamma2, beta2, mean2, var2, w_lin, b_lin):
    B, T, C = x.shape
    F = _F
    T2 = T - (_KT - 1)
    P1 = (T2 - _P) // _P + 1
    T3 = P1 - (_KT - 1)
    P2 = (T3 - _P) // _P + 1
    E = w_lin.shape[0]
    M1 = P1 + _NS - 1
    M2 = P2 + _NS - 1

    s1 = gamma1 * jax.lax.rsqrt(var1 + _EPS)
    t1 = beta1 - mean1 * s1
    s2 = gamma2 * jax.lax.rsqrt(var2 + _EPS)
    t2 = beta2 - mean2 * s2

    a_i = jnp.arange(_NS)[:, None, None]
    s_i = jnp.arange(_P)[None, :, None]
    r_i = jnp.arange(_P)[None, None, :]
    tap = _P * a_i + s_i - r_i
    ok = (tap >= 0) & (tap < _KT)
    tap_c = jnp.clip(tap, 0, _KT - 1)

    win_tail = (_P - 1 + _KT - 1 + 1) - (_NS - 1) * _P
    ka = ((_NS - 1) * _P + win_tail) * C
    kb = ((_NS - 1) * _P + win_tail) * F

    w2_band = jnp.where(ok[..., None], w2.T[tap_c], 0.0)
    w_a = jnp.einsum("ci,asrf->asircf", w1, w2_band)
    w_a = w_a.reshape(_NS * _P * C, _P * C * F)[:ka].astype(jnp.bfloat16)
    bias_a = (b1[:, None] * jnp.sum(w2, axis=1)[None, :]
              + b2[None, :]).reshape(-1)
    aff_a = jnp.stack([bias_a, jnp.tile(s1, C), jnp.tile(t1, C)], axis=0)

    w3_band = jnp.where(ok[..., None, None],
                        jnp.transpose(w3, (2, 1, 0))[tap_c], 0.0)
    w_b = (jnp.transpose(w3_band, (0, 1, 3, 2, 4))
           .reshape(_NS * _P * F, _P * F)[:kb].astype(jnp.bfloat16))
    aff_b = jnp.stack([b3, s2, t2], axis=0)

    w_l = (w_lin.reshape(E, F, P2, C).transpose(3, 2, 1, 0)
           .reshape(C * P2 * F, E).astype(jnp.bfloat16))
    b_l = b_lin.reshape(1, E)

    xb = (jnp.pad(x, ((0, 0), (0, M1 * _P - T), (0, 0)))
          .reshape(B, M1, _P * C).astype(jnp.bfloat16))

    ga = 16
    while B % ga:
        ga //= 2
    p1r = ((M2 * _P + 7) // 8) * 8
    m2r = ((M2 + 7) // 8) * 8
    s_t = jnp.repeat(jnp.arange(_P), M2)
    m_t = jnp.tile(jnp.arange(M2), _P)
    perm = (jnp.zeros((_P * m2r, p1r), jnp.bfloat16)
            .at[s_t * m2r + m_t, _P * m_t + s_t].set(1))
    z2 = pl.pallas_call(
        _conv_body,
        out_shape=jax.ShapeDtypeStruct((B, C * P2, F), jnp.bfloat16),
        grid=(B // ga,),
        in_specs=[
            pl.BlockSpec((ga, M1, _P * C), lambda i: (i, 0, 0)),
            pl.BlockSpec(w_a.shape, lambda i: (0, 0)),
            pl.BlockSpec(aff_a.shape, lambda i: (0, 0)),
            pl.BlockSpec(w_b.shape, lambda i: (0, 0)),
            pl.BlockSpec(aff_b.shape, lambda i: (0, 0)),
            pl.BlockSpec(perm.shape, lambda i: (0, 0)),
        ],
        out_specs=pl.BlockSpec((ga, C * P2, F), lambda i: (i, 0, 0)),
        scratch_shapes=[
            pltpu.VMEM((ga * p1r, ka), jnp.bfloat16),
            pltpu.VMEM((ga * C * m2r, _P * F), jnp.bfloat16),
            pltpu.VMEM((ga * C * P2, kb), jnp.bfloat16),
        ],
        compiler_params=pltpu.CompilerParams(
            dimension_semantics=("parallel",)),
    )(xb, w_a, aff_a, w_b, aff_b, perm)

    zf = z2.reshape(B, C * P2 * F)

    bb = min(B, 128)
    out = pl.pallas_call(
        _head_body,
        out_shape=jax.ShapeDtypeStruct((B, E), jnp.float32),
        grid=(pl.cdiv(B, bb),),
        in_specs=[
            pl.BlockSpec((bb, C * P2 * F), lambda i: (i, 0)),
            pl.BlockSpec(w_l.shape, lambda i: (0, 0)),
            pl.BlockSpec(b_l.shape, lambda i: (0, 0)),
        ],
        out_specs=pl.BlockSpec((bb, E), lambda i: (i, 0)),
        compiler_params=pltpu.CompilerParams(
            dimension_semantics=("parallel",)),
    )(zf, w_l, b_l)
    return out


# speedup vs baseline: 1.2463x; 1.0289x over previous
import jax
import jax.numpy as jnp
from jax.experimental import pallas as pl
from jax.experimental.pallas import tpu as pltpu

_F = 16
_KT = 50
_P = 13
_EPS = 1e-5
_NS = (_P - 1 + _KT - 1) // _P + 1


def _conv_body(x_ref, wa_ref, affa_ref, wb_ref, affb_ref, perm_ref, o_ref,
               xw_ref, zb_ref, zw_ref):
    ga = o_ref.shape[0]
    kc = x_ref.shape[2]
    c = kc // _P
    f = o_ref.shape[2]
    na = c * f
    kf = _P * f
    p2 = o_ref.shape[1] // c
    m2 = p2 + _NS - 1
    p1 = m2 * _P
    zbs = zb_ref.shape[0] // (ga * c)
    m2r = perm_ref.shape[0] // _P
    rows = xw_ref.shape[0] // ga
    tail_a = xw_ref.shape[1] - (_NS - 1) * kc
    tail_b = zw_ref.shape[1] - (_NS - 1) * kf

    for g in range(ga):
        r0 = g * rows
        for a in range(_NS - 1):
            xw_ref[r0:r0 + p1, a * kc:(a + 1) * kc] = \
                x_ref[g, a:a + p1, :].astype(xw_ref.dtype)
        xw_ref[r0:r0 + p1, (_NS - 1) * kc:] = \
            x_ref[g, _NS - 1:_NS - 1 + p1, 0:tail_a].astype(xw_ref.dtype)
        xw_ref[r0 + p1:r0 + rows, :] = jnp.zeros_like(xw_ref[r0 + p1:r0 + rows])

    chunks = []
    rem = _P
    while rem > 3:
        chunks.append(2)
        rem -= 2
    chunks.append(rem)
    for g in range(ga):
        xg = xw_ref[g * rows:(g + 1) * rows, :]
        m = None
        n0 = 0
        for k in chunks:
            d = jnp.dot(xg, wa_ref[:, n0:n0 + k * na],
                        preferred_element_type=jnp.float32)
            for l in range(k):
                piece = d[:, l * na:(l + 1) * na]
                m = piece if m is None else jnp.maximum(m, piece)
            n0 += k * na
        y = jnp.maximum(m + affa_ref[0:1], 0.0) * affa_ref[1:2] + affa_ref[2:3]
        y = y.astype(zb_ref.dtype)

        yp = jnp.dot(perm_ref[...], y,
                     preferred_element_type=jnp.float32).astype(zb_ref.dtype)
        for ci in range(c):
            zb0 = (g * c + ci) * zbs
            for s in range(_P):
                zb_ref[zb0:zb0 + m2, s * f:(s + 1) * f] = \
                    yp[s * m2r:s * m2r + m2, ci * f:(ci + 1) * f]

    for gc in range(ga * c):
        r0 = gc * p2
        z0 = gc * zbs
        for a in range(_NS - 1):
            zw_ref[r0:r0 + p2, a * kf:(a + 1) * kf] = \
                zb_ref[z0 + a:z0 + a + p2, :]
        zw_ref[r0:r0 + p2, (_NS - 1) * kf:] = \
            zb_ref[z0 + _NS - 1:z0 + _NS - 1 + p2, 0:tail_b]

    for g in range(ga):
        accb = jnp.dot(zw_ref[g * c * p2:(g + 1) * c * p2, :], wb_ref[...],
                       preferred_element_type=jnp.float32)
        mb = accb[:, 0:f]
        for r in range(1, _P):
            mb = jnp.maximum(mb, accb[:, r * f:(r + 1) * f])
        z2 = (jnp.maximum(mb + affb_ref[0:1], 0.0) * affb_ref[1:2]
              + affb_ref[2:3])
        o_ref[g] = z2.astype(o_ref.dtype)


def _head_body(z_ref, w_ref, b_ref, o_ref):
    o_ref[...] = (jnp.dot(z_ref[...], w_ref[...],
                          preferred_element_type=jnp.float32) + b_ref[...])


def kernel(x, w1, b1, w2, b2, w3, b3, gamma1, beta1, mean1, var1,
           gamma2, beta2, mean2, var2, w_lin, b_lin):
    B, T, C = x.shape
    F = _F
    T2 = T - (_KT - 1)
    P1 = (T2 - _P) // _P + 1
    T3 = P1 - (_KT - 1)
    P2 = (T3 - _P) // _P + 1
    E = w_lin.shape[0]
    M1 = P1 + _NS - 1
    M2 = P2 + _NS - 1

    s1 = gamma1 * jax.lax.rsqrt(var1 + _EPS)
    t1 = beta1 - mean1 * s1
    s2 = gamma2 * jax.lax.rsqrt(var2 + _EPS)
    t2 = beta2 - mean2 * s2

    a_i = jnp.arange(_NS)[:, None, None]
    s_i = jnp.arange(_P)[None, :, None]
    r_i = jnp.arange(_P)[None, None, :]
    tap = _P * a_i + s_i - r_i
    ok = (tap >= 0) & (tap < _KT)
    tap_c = jnp.clip(tap, 0, _KT - 1)

    win_tail = (_P - 1 + _KT - 1 + 1) - (_NS - 1) * _P
    ka = ((_NS - 1) * _P + win_tail) * C
    kb = ((_NS - 1) * _P + win_tail) * F

    w2_band = jnp.where(ok[..., None], w2.T[tap_c], 0.0)
    w_a = jnp.einsum("ci,asrf->asircf", w1, w2_band)
    w_a = w_a.reshape(_NS * _P * C, _P * C * F)[:ka].astype(jnp.bfloat16)
    bias_a = (b1[:, None] * jnp.sum(w2, axis=1)[None, :]
              + b2[None, :]).reshape(-1)
    aff_a = jnp.stack([bias_a, jnp.tile(s1, C), jnp.tile(t1, C)], axis=0)

    w3_band = jnp.where(ok[..., None, None],
                        jnp.transpose(w3, (2, 1, 0))[tap_c], 0.0)
    w_b = (jnp.transpose(w3_band, (0, 1, 3, 2, 4))
           .reshape(_NS * _P * F, _P * F)[:kb].astype(jnp.bfloat16))
    aff_b = jnp.stack([b3, s2, t2], axis=0)

    w_l = (w_lin.reshape(E, F, P2, C).transpose(3, 2, 1, 0)
           .reshape(C * P2 * F, E).astype(jnp.bfloat16))
    b_l = b_lin.reshape(1, E)

    xb = (jnp.pad(x, ((0, 0), (0, M1 * _P - T), (0, 0)))
          .reshape(B, M1, _P * C))

    ga = 16
    while B % ga:
        ga //= 2
    p1r = ((M2 * _P + 7) // 8) * 8
    m2r = ((M2 + 7) // 8) * 8
    s_t = jnp.repeat(jnp.arange(_P), M2)
    m_t = jnp.tile(jnp.arange(M2), _P)
    perm = (jnp.zeros((_P * m2r, p1r), jnp.bfloat16)
            .at[s_t * m2r + m_t, _P * m_t + s_t].set(1))
    z2 = pl.pallas_call(
        _conv_body,
        out_shape=jax.ShapeDtypeStruct((B, C * P2, F), jnp.bfloat16),
        grid=(B // ga,),
        in_specs=[
            pl.BlockSpec((ga, M1, _P * C), lambda i: (i, 0, 0)),
            pl.BlockSpec(w_a.shape, lambda i: (0, 0)),
            pl.BlockSpec(aff_a.shape, lambda i: (0, 0)),
            pl.BlockSpec(w_b.shape, lambda i: (0, 0)),
            pl.BlockSpec(aff_b.shape, lambda i: (0, 0)),
            pl.BlockSpec(perm.shape, lambda i: (0, 0)),
        ],
        out_specs=pl.BlockSpec((ga, C * P2, F), lambda i: (i, 0, 0)),
        scratch_shapes=[
            pltpu.VMEM((ga * p1r, ka), jnp.bfloat16),
            pltpu.VMEM((ga * C * m2r, _P * F), jnp.bfloat16),
            pltpu.VMEM((ga * C * P2, kb), jnp.bfloat16),
        ],
        compiler_params=pltpu.CompilerParams(
            dimension_semantics=("parallel",)),
    )(xb, w_a, aff_a, w_b, aff_b, perm)

    zf = z2.reshape(B, C * P2 * F)

    bb = min(B, 512)
    out = pl.pallas_call(
        _head_body,
        out_shape=jax.ShapeDtypeStruct((B, E), jnp.float32),
        grid=(pl.cdiv(B, bb),),
        in_specs=[
            pl.BlockSpec((bb, C * P2 * F), lambda i: (i, 0)),
            pl.BlockSpec(w_l.shape, lambda i: (0, 0)),
            pl.BlockSpec(b_l.shape, lambda i: (0, 0)),
        ],
        out_specs=pl.BlockSpec((bb, E), lambda i: (i, 0)),
        compiler_params=pltpu.CompilerParams(
            dimension_semantics=("parallel",)),
    )(zf, w_l, b_l)
    return out
